# initial kernel scaffold (unmeasured)
import jax
import jax.numpy as jnp
from jax import lax
from jax.experimental import pallas as pl
from jax.experimental.pallas import tpu as pltpu

N_DEV = 4
HQ = 32
HG = 8
NRES = 4
NQB = 8
BLK = 64
SEQ_R = NQB * BLK
SQ = 2048
DM = 1024
DQK = 1024
DH = 128
SCALE = 0.08838834764831843
F32 = jnp.float32


def kernel(x, Wq, K_ext, V_ext, Wo):
    x4 = x.reshape(NQB, NRES, BLK, DM)
    K5 = K_ext.reshape(NQB, NRES, BLK, HQ, DH)
    V5 = V_ext.reshape(NQB, NRES, BLK, HQ, DH)

    def body(x_ref, wq_ref, k_ref, v_ref, wo_ref, out_ref,
             x_scr, comm, q_scr, ctx_scr, k_scr, v_scr,
             sem_x, sem_k, sem_v, send_sems, recv_sems):
        my = lax.axis_index("i")
        left = (my + N_DEV - 1) % N_DEV
        right = (my + 1) % N_DEV

        barrier_sem = pltpu.get_barrier_semaphore()
        for nbr in (left, right):
            pl.semaphore_signal(
                barrier_sem, inc=1,
                device_id=(nbr,), device_id_type=pl.DeviceIdType.MESH,
            )
        pl.semaphore_wait(barrier_sem, 2)

        x_cps = [
            pltpu.make_async_copy(x_ref.at[:, r], x_scr.at[r], sem_x)
            for r in range(NRES)
        ]
        for c in x_cps:
            c.start()

        comm[0, 0] = wq_ref[:]
        comm[0, 1] = wo_ref[:]

        for c in x_cps:
            c.wait()

        for h in range(N_DEV):
            g = (my + N_DEV - h) % N_DEV
            base = g * HG

            kv_cps = []
            for hh in range(HG):
                for r in range(NRES):
                    kv_cps.append(pltpu.make_async_copy(
                        k_ref.at[:, r, :, base + hh], k_scr.at[hh, r], sem_k))
                    kv_cps.append(pltpu.make_async_copy(
                        v_ref.at[:, r, :, base + hh], v_scr.at[hh, r], sem_v))
            for c in kv_cps:
                c.start()

            if h < N_DEV - 1:
                rdma = pltpu.make_async_remote_copy(
                    src_ref=comm.at[h],
                    dst_ref=comm.at[h + 1],
                    send_sem=send_sems.at[h],
                    recv_sem=recv_sems.at[h],
                    device_id=(right,),
                    device_id_type=pl.DeviceIdType.MESH,
                )
                rdma.start()

            q_scr[:] = jnp.dot(
                x_scr[:].reshape(SQ, DM), comm[h, 0],
                preferred_element_type=F32,
            )

            for c in kv_cps:
                c.wait()

            for hh in range(HG):
                q3 = q_scr[:, hh * DH:(hh + 1) * DH].reshape(NRES, SEQ_R, DH)
                k3 = k_scr[hh].reshape(NRES, SEQ_R, DH)
                v3 = v_scr[hh].reshape(NRES, SEQ_R, DH)
                s = lax.dot_general(
                    q3, k3, (((2,), (2,)), ((0,), (0,))),
                    preferred_element_type=F32,
                ) * SCALE
                m = jnp.max(s, axis=2, keepdims=True)
                e = jnp.exp(s - m)
                w = e / jnp.sum(e, axis=2, keepdims=True)
                ctx3 = lax.dot_general(
                    w, v3, (((2,), (1,)), ((0,), (0,))),
                    preferred_element_type=F32,
                )
                ctx_scr[:, hh * DH:(hh + 1) * DH] = ctx3.reshape(SQ, DH)

            part = jnp.dot(ctx_scr[:], comm[h, 1], preferred_element_type=F32)
            if h == 0:
                out_ref[:] = part
            else:
                out_ref[:] = out_ref[:] + part

            if h < N_DEV - 1:
                rdma.wait()

    out_perm = pl.pallas_call(
        body,
        out_shape=jax.ShapeDtypeStruct((SQ, DQK), F32),
        in_specs=[
            pl.BlockSpec(memory_space=pl.ANY),
            pl.BlockSpec(memory_space=pltpu.VMEM),
            pl.BlockSpec(memory_space=pl.ANY),
            pl.BlockSpec(memory_space=pl.ANY),
            pl.BlockSpec(memory_space=pltpu.VMEM),
        ],
        out_specs=pl.BlockSpec(memory_space=pltpu.VMEM),
        scratch_shapes=[
            pltpu.VMEM((NRES, NQB, BLK, DM), F32),
            pltpu.VMEM((N_DEV, 2, DM, DQK), F32),
            pltpu.VMEM((SQ, DQK), F32),
            pltpu.VMEM((SQ, DQK), F32),
            pltpu.VMEM((HG, NRES, NQB, BLK, DH), F32),
            pltpu.VMEM((HG, NRES, NQB, BLK, DH), F32),
            pltpu.SemaphoreType.DMA,
            pltpu.SemaphoreType.DMA,
            pltpu.SemaphoreType.DMA,
            pltpu.SemaphoreType.DMA((N_DEV - 1,)),
            pltpu.SemaphoreType.DMA((N_DEV - 1,)),
        ],
        compiler_params=pltpu.CompilerParams(collective_id=0),
    )(x4, Wq, K5, V5, Wo)

    return (
        out_perm.reshape(NRES, NQB, BLK, DM)
        .transpose(1, 0, 2, 3)
        .reshape(1, SQ, DM)
    )


# baseline (device time: 316970 ns/iter reference)
import jax
import jax.numpy as jnp
from jax import lax
from jax.experimental import pallas as pl
from jax.experimental.pallas import tpu as pltpu

N_DEV = 4
HQ = 32
HG = 8
NRES = 4
NQB = 8
BLK = 64
SEQ_R = NQB * BLK
SQ = 2048
DM = 1024
DQK = 1024
DH = 128
NCHUNK = N_DEV * NRES
SCALE = 0.08838834764831843
F32 = jnp.float32


def kernel(x, Wq, K_ext, V_ext, Wo):
    x4 = x.reshape(NQB, NRES, BLK, DM)
    K5 = K_ext.reshape(NQB, NRES, BLK, HQ, DH)
    V5 = V_ext.reshape(NQB, NRES, BLK, HQ, DH)

    def body(x_ref, wq_ref, k_ref, v_ref, wo_ref, out_ref,
             x_scr, comm, ctx_scr, k_scr, v_scr,
             sem_x, k_sems, v_sems, send_sems, recv_sems, credit_sem):
        my = lax.axis_index("i")
        left = (my + N_DEV - 1) % N_DEV
        right = (my + 1) % N_DEV

        x_cps = [
            pltpu.make_async_copy(x_ref.at[:, r], x_scr.at[r], sem_x)
            for r in range(NRES)
        ]
        for cp in x_cps:
            cp.start()

        def kv_copies(c):
            h, r = divmod(c, NRES)
            g = (my + N_DEV - h) % N_DEV
            buf = c % 2
            cps = []
            for hh in range(HG):
                head = g * HG + hh
                cps.append(pltpu.make_async_copy(
                    k_ref.at[:, r, :, head], k_scr.at[buf, hh],
                    k_sems.at[buf]))
                cps.append(pltpu.make_async_copy(
                    v_ref.at[:, r, :, head], v_scr.at[buf, hh],
                    v_sems.at[buf]))
            return cps

        for c0 in (0, 1):
            for cp in kv_copies(c0):
                cp.start()

        barrier_sem = pltpu.get_barrier_semaphore()
        for nbr in (left, right):
            pl.semaphore_signal(
                barrier_sem, inc=1,
                device_id=(nbr,), device_id_type=pl.DeviceIdType.MESH,
            )
        pl.semaphore_wait(barrier_sem, 2)

        comm[0, 0] = wq_ref[:]
        comm[0, 1] = wo_ref[:]

        for cp in x_cps:
            cp.wait()

        for h in range(N_DEV):
            s_slot = h % 2
            if h < N_DEV - 1:
                if h >= 1:
                    pl.semaphore_wait(credit_sem, 1)
                rdma = pltpu.make_async_remote_copy(
                    src_ref=comm.at[s_slot],
                    dst_ref=comm.at[1 - s_slot],
                    send_sem=send_sems.at[h],
                    recv_sem=recv_sems.at[h],
                    device_id=(right,),
                    device_id_type=pl.DeviceIdType.MESH,
                )
                rdma.start()

            for r in range(NRES):
                c = h * NRES + r
                buf = c % 2
                for cp in kv_copies(c):
                    cp.wait()
                q_r = jnp.dot(
                    x_scr[r].reshape(SEQ_R, DM), comm[s_slot, 0],
                    preferred_element_type=F32,
                )
                for hh in range(HG):
                    q_h = q_r[:, hh * DH:(hh + 1) * DH]
                    k_h = k_scr[buf, hh].reshape(SEQ_R, DH)
                    v_h = v_scr[buf, hh].reshape(SEQ_R, DH)
                    s = lax.dot_general(
                        q_h, k_h, (((1,), (1,)), ((), ())),
                        preferred_element_type=F32,
                    ) * SCALE
                    m = jnp.max(s, axis=1, keepdims=True)
                    e = jnp.exp(s - m)
                    w = e / jnp.sum(e, axis=1, keepdims=True)
                    ctx_scr[:, hh * DH:(hh + 1) * DH] = jnp.dot(
                        w, v_h, preferred_element_type=F32)
                part = jnp.dot(
                    ctx_scr[:], comm[s_slot, 1], preferred_element_type=F32)
                row = slice(r * SEQ_R, (r + 1) * SEQ_R)
                if h == 0:
                    out_ref[row, :] = part
                else:
                    out_ref[row, :] = out_ref[row, :] + part
                if c + 2 < NCHUNK:
                    for cp in kv_copies(c + 2):
                        cp.start()

            if h < N_DEV - 2:
                pl.semaphore_signal(
                    credit_sem, inc=1,
                    device_id=(left,), device_id_type=pl.DeviceIdType.MESH,
                )
            if h < N_DEV - 1:
                rdma.wait()

    out_perm = pl.pallas_call(
        body,
        out_shape=jax.ShapeDtypeStruct((SQ, DQK), F32),
        in_specs=[
            pl.BlockSpec(memory_space=pl.ANY),
            pl.BlockSpec(memory_space=pltpu.VMEM),
            pl.BlockSpec(memory_space=pl.ANY),
            pl.BlockSpec(memory_space=pl.ANY),
            pl.BlockSpec(memory_space=pltpu.VMEM),
        ],
        out_specs=pl.BlockSpec(memory_space=pltpu.VMEM),
        scratch_shapes=[
            pltpu.VMEM((NRES, NQB, BLK, DM), F32),
            pltpu.VMEM((2, 2, DM, DQK), F32),
            pltpu.VMEM((SEQ_R, DQK), F32),
            pltpu.VMEM((2, HG, NQB, BLK, DH), F32),
            pltpu.VMEM((2, HG, NQB, BLK, DH), F32),
            pltpu.SemaphoreType.DMA,
            pltpu.SemaphoreType.DMA((2,)),
            pltpu.SemaphoreType.DMA((2,)),
            pltpu.SemaphoreType.DMA((N_DEV - 1,)),
            pltpu.SemaphoreType.DMA((N_DEV - 1,)),
            pltpu.SemaphoreType.REGULAR,
        ],
        compiler_params=pltpu.CompilerParams(
            collective_id=0,
            vmem_limit_bytes=64 * 1024 * 1024,
        ),
    )(x4, Wq, K5, V5, Wo)

    return (
        out_perm.reshape(NRES, NQB, BLK, DM)
        .transpose(1, 0, 2, 3)
        .reshape(1, SQ, DM)
    )


# device time: 184143 ns/iter; 1.7213x vs baseline; 1.7213x over previous
import jax
import jax.numpy as jnp
from jax import lax
from jax.experimental import pallas as pl
from jax.experimental.pallas import tpu as pltpu

N_DEV = 4
HQ = 32
HG = 8
NRES = 4
NQB = 8
BLK = 64
SEQ_R = NQB * BLK
SQ = 2048
DM = 1024
DQK = 1024
DH = 128
NCHUNK = N_DEV * NRES
SCALE = 0.08838834764831843
F32 = jnp.float32
BF16 = jnp.bfloat16


def kernel(x, Wq, K_ext, V_ext, Wo):
    x4 = x.reshape(NQB, NRES, BLK, DM)
    K5 = K_ext.reshape(NQB, NRES, BLK, HQ, DH)
    V5 = V_ext.reshape(NQB, NRES, BLK, HQ, DH)

    def body(x_ref, wq_ref, k_ref, v_ref, wo_ref, out_ref,
             x_scr, x_bf, comm, ctx_scr, k_scr, v_scr,
             sem_x, k_sems, v_sems, send_sems, recv_sems, credit_sem):
        my = lax.axis_index("i")
        left = (my + N_DEV - 1) % N_DEV
        right = (my + 1) % N_DEV

        x_cps = [
            pltpu.make_async_copy(x_ref.at[:, r], x_scr.at[r], sem_x)
            for r in range(NRES)
        ]
        for cp in x_cps:
            cp.start()

        def kv_copies(c):
            h, r = divmod(c, NRES)
            g = (my + N_DEV - h) % N_DEV
            buf = c % 2
            cps = []
            for hh in range(HG):
                head = g * HG + hh
                cps.append(pltpu.make_async_copy(
                    k_ref.at[:, r, :, head], k_scr.at[buf, hh],
                    k_sems.at[buf]))
                cps.append(pltpu.make_async_copy(
                    v_ref.at[:, r, :, head], v_scr.at[buf, hh],
                    v_sems.at[buf]))
            return cps

        for c0 in (0, 1):
            for cp in kv_copies(c0):
                cp.start()

        barrier_sem = pltpu.get_barrier_semaphore()
        for nbr in (left, right):
            pl.semaphore_signal(
                barrier_sem, inc=1,
                device_id=(nbr,), device_id_type=pl.DeviceIdType.MESH,
            )
        pl.semaphore_wait(barrier_sem, 2)

        comm[0, 0] = wq_ref[:].astype(BF16)
        comm[0, 1] = wo_ref[:].astype(BF16)

        for cp in x_cps:
            cp.wait()
        x_bf[:] = x_scr[:].reshape(SQ, DM).astype(BF16)

        for h in range(N_DEV):
            s_slot = h % 2
            if h < N_DEV - 1:
                if h >= 1:
                    pl.semaphore_wait(credit_sem, 1)
                rdma = pltpu.make_async_remote_copy(
                    src_ref=comm.at[s_slot],
                    dst_ref=comm.at[1 - s_slot],
                    send_sem=send_sems.at[h],
                    recv_sem=recv_sems.at[h],
                    device_id=(right,),
                    device_id_type=pl.DeviceIdType.MESH,
                )
                rdma.start()

            for r in range(NRES):
                c = h * NRES + r
                buf = c % 2
                for cp in kv_copies(c):
                    cp.wait()
                q_r = jnp.dot(
                    x_bf[r * SEQ_R:(r + 1) * SEQ_R, :], comm[s_slot, 0],
                    preferred_element_type=F32,
                ).astype(BF16)
                for hh in range(HG):
                    q_h = q_r[:, hh * DH:(hh + 1) * DH]
                    k_h = k_scr[buf, hh].reshape(SEQ_R, DH).astype(BF16)
                    v_h = v_scr[buf, hh].reshape(SEQ_R, DH).astype(BF16)
                    s = lax.dot_general(
                        q_h, k_h, (((1,), (1,)), ((), ())),
                        preferred_element_type=F32,
                    ) * SCALE
                    m = jnp.max(s, axis=1, keepdims=True)
                    e = jnp.exp(s - m)
                    w = (e / jnp.sum(e, axis=1, keepdims=True)).astype(BF16)
                    ctx_scr[:, hh * DH:(hh + 1) * DH] = jnp.dot(
                        w, v_h, preferred_element_type=F32).astype(BF16)
                part = jnp.dot(
                    ctx_scr[:], comm[s_slot, 1], preferred_element_type=F32)
                row = slice(r * SEQ_R, (r + 1) * SEQ_R)
                if h == 0:
                    out_ref[row, :] = part
                else:
                    out_ref[row, :] = out_ref[row, :] + part
                if c + 2 < NCHUNK:
                    for cp in kv_copies(c + 2):
                        cp.start()

            if h < N_DEV - 2:
                pl.semaphore_signal(
                    credit_sem, inc=1,
                    device_id=(left,), device_id_type=pl.DeviceIdType.MESH,
                )
            if h < N_DEV - 1:
                rdma.wait()

    out_perm = pl.pallas_call(
        body,
        out_shape=jax.ShapeDtypeStruct((SQ, DQK), F32),
        in_specs=[
            pl.BlockSpec(memory_space=pl.ANY),
            pl.BlockSpec(memory_space=pltpu.VMEM),
            pl.BlockSpec(memory_space=pl.ANY),
            pl.BlockSpec(memory_space=pl.ANY),
            pl.BlockSpec(memory_space=pltpu.VMEM),
        ],
        out_specs=pl.BlockSpec(memory_space=pltpu.VMEM),
        scratch_shapes=[
            pltpu.VMEM((NRES, NQB, BLK, DM), F32),
            pltpu.VMEM((SQ, DM), BF16),
            pltpu.VMEM((2, 2, DM, DQK), BF16),
            pltpu.VMEM((SEQ_R, DQK), BF16),
            pltpu.VMEM((2, HG, NQB, BLK, DH), F32),
            pltpu.VMEM((2, HG, NQB, BLK, DH), F32),
            pltpu.SemaphoreType.DMA,
            pltpu.SemaphoreType.DMA((2,)),
            pltpu.SemaphoreType.DMA((2,)),
            pltpu.SemaphoreType.DMA((N_DEV - 1,)),
            pltpu.SemaphoreType.DMA((N_DEV - 1,)),
            pltpu.SemaphoreType.REGULAR,
        ],
        compiler_params=pltpu.CompilerParams(
            collective_id=0,
            vmem_limit_bytes=64 * 1024 * 1024,
        ),
    )(x4, Wq, K5, V5, Wo)

    return (
        out_perm.reshape(NRES, NQB, BLK, DM)
        .transpose(1, 0, 2, 3)
        .reshape(1, SQ, DM)
    )
